# Initial kernel scaffold; baseline (speedup 1.0000x reference)
import jax
import jax.numpy as jnp
from jax.experimental import pallas as pl


def kernel(node_features, edge_index, edge_features, node_W, node_b, edge_W, edge_b, msg_W0, msg_b0, upd_W0, upd_b0, ln_g0, ln_b0, msg_W1, msg_b1, upd_W1, upd_b1, ln_g1, ln_b1, msg_W2, msg_b2, upd_W2, upd_b2, ln_g2, ln_b2, score_W, score_b):
    raise NotImplementedError("write your pallas kernel here")


# speedup vs baseline: 1.9730x; 1.9730x over previous
import functools

import jax
import jax.numpy as jnp
from jax import lax
from jax.experimental import pallas as pl
from jax.experimental.pallas import tpu as pltpu
from jax.experimental.pallas import tpu_sc as plsc

N_NODES = 10000
N_EDGES = 160000
D_IN = 256
H = 512

NC = 2
NS = 16
LANES = 16
CHUNK = 128
NCHUNK = H // CHUNK
CHUNKS_PER_CORE = NCHUNK // NC
EDGES_PER_TILE = N_EDGES // NS
EB = 80
NBATCH = EDGES_PER_TILE // EB
RPT = 624
RTAIL = N_NODES - RPT * NS

BM = 400
BE = 400

_EPS = 1e-5


def _relu_mm_kernel(x_ref, w_ref, b_ref, o_ref):
  o_ref[:] = jnp.maximum(
      jnp.dot(x_ref[:], w_ref[:], preferred_element_type=jnp.float32)
      + b_ref[:], 0.0)


def _node_proj(x, w, b):
  return pl.pallas_call(
      _relu_mm_kernel,
      grid=(N_NODES // BM,),
      in_specs=[
          pl.BlockSpec((BM, D_IN), lambda i: (i, 0)),
          pl.BlockSpec((D_IN, H), lambda i: (0, 0)),
          pl.BlockSpec((1, H), lambda i: (0, 0)),
      ],
      out_specs=pl.BlockSpec((BM, H), lambda i: (i, 0)),
      out_shape=jax.ShapeDtypeStruct((N_NODES, H), jnp.float32),
  )(x, w, b)


def _edge_msg_kernel(ef_ref, ew_ref, eb_ref, w0_ref, w1_ref, w2_ref,
                     o0_ref, o1_ref, o2_ref):
  e = jnp.maximum(
      jnp.dot(ef_ref[:], ew_ref[:], preferred_element_type=jnp.float32)
      + eb_ref[:], 0.0)
  for w_ref, o_ref in ((w0_ref, o0_ref), (w1_ref, o1_ref), (w2_ref, o2_ref)):
    em = jnp.dot(e, w_ref[:], preferred_element_type=jnp.float32)
    for k in range(NCHUNK):
      o_ref[k] = em[:, k * CHUNK:(k + 1) * CHUNK]


def _edge_msg(ef, ew, eb, we0, we1, we2):
  out = jax.ShapeDtypeStruct((NCHUNK, N_EDGES, CHUNK), jnp.float32)
  return pl.pallas_call(
      _edge_msg_kernel,
      grid=(N_EDGES // BE,),
      in_specs=[
          pl.BlockSpec((BE, D_IN), lambda i: (i, 0)),
          pl.BlockSpec((D_IN, H), lambda i: (0, 0)),
          pl.BlockSpec((1, H), lambda i: (0, 0)),
          pl.BlockSpec((H, H), lambda i: (0, 0)),
          pl.BlockSpec((H, H), lambda i: (0, 0)),
          pl.BlockSpec((H, H), lambda i: (0, 0)),
      ],
      out_specs=[pl.BlockSpec((NCHUNK, BE, CHUNK), lambda i: (0, i, 0))] * 3,
      out_shape=[out, out, out],
  )(ef, ew, eb, we0, we1, we2)


def _ac_kernel(h_ref, wj_ref, wi_ref, mb_ref, a_ref, c_ref):
  a = jnp.dot(h_ref[:], wj_ref[:], preferred_element_type=jnp.float32) + mb_ref[:]
  c = jnp.dot(h_ref[:], wi_ref[:], preferred_element_type=jnp.float32)
  for k in range(NCHUNK):
    sl = slice(k * CHUNK, (k + 1) * CHUNK)
    a_ref[k] = a[:, sl]
    c_ref[k] = c[:, sl]


def _ac_proj(h, wj, wi, mb):
  out = jax.ShapeDtypeStruct((NCHUNK, N_NODES, CHUNK), jnp.float32)
  return pl.pallas_call(
      _ac_kernel,
      grid=(N_NODES // BM,),
      in_specs=[
          pl.BlockSpec((BM, H), lambda i: (i, 0)),
          pl.BlockSpec((H, H), lambda i: (0, 0)),
          pl.BlockSpec((H, H), lambda i: (0, 0)),
          pl.BlockSpec((1, H), lambda i: (0, 0)),
      ],
      out_specs=[pl.BlockSpec((NCHUNK, BM, CHUNK), lambda i: (0, i, 0))] * 2,
      out_shape=[out, out],
  )(h, wj, wi, mb)


def _layer_norm(u, g_ref, b_ref):
  mu = jnp.mean(u, axis=-1, keepdims=True)
  var = jnp.mean((u - mu) ** 2, axis=-1, keepdims=True)
  return (u - mu) / jnp.sqrt(var + _EPS) * g_ref[:] + b_ref[:]


def _update_kernel(h_ref, ag_ref, wt_ref, wb_ref, ub_ref, g_ref, b_ref, o_ref):
  acc = jnp.dot(h_ref[:], wt_ref[:], preferred_element_type=jnp.float32) + ub_ref[:]
  for k in range(NCHUNK):
    acc += jnp.dot(ag_ref[k], wb_ref[k], preferred_element_type=jnp.float32)
  o_ref[:] = _layer_norm(jnp.maximum(acc, 0.0), g_ref, b_ref)


def _update_score_kernel(h_ref, ag_ref, wt_ref, wb_ref, ub_ref, g_ref, b_ref,
                         sw_ref, sb_ref, o_ref, s_ref):
  acc = jnp.dot(h_ref[:], wt_ref[:], preferred_element_type=jnp.float32) + ub_ref[:]
  for k in range(NCHUNK):
    acc += jnp.dot(ag_ref[k], wb_ref[k], preferred_element_type=jnp.float32)
  hn = _layer_norm(jnp.maximum(acc, 0.0), g_ref, b_ref)
  o_ref[:] = hn
  s_ref[:] = jnp.dot(hn, sw_ref[:], preferred_element_type=jnp.float32) + sb_ref[:]


_UPD_IN_SPECS = [
    pl.BlockSpec((BM, H), lambda i: (i, 0)),
    pl.BlockSpec((NCHUNK, BM, CHUNK), lambda i: (0, i, 0)),
    pl.BlockSpec((H, H), lambda i: (0, 0)),
    pl.BlockSpec((NCHUNK, CHUNK, H), lambda i: (0, 0, 0)),
    pl.BlockSpec((1, H), lambda i: (0, 0)),
    pl.BlockSpec((1, H), lambda i: (0, 0)),
    pl.BlockSpec((1, H), lambda i: (0, 0)),
]


def _update(h, ag, wt, wb4, ub, g, b):
  return pl.pallas_call(
      _update_kernel,
      grid=(N_NODES // BM,),
      in_specs=list(_UPD_IN_SPECS),
      out_specs=pl.BlockSpec((BM, H), lambda i: (i, 0)),
      out_shape=jax.ShapeDtypeStruct((N_NODES, H), jnp.float32),
  )(h, ag, wt, wb4, ub, g, b)


def _update_score(h, ag, wt, wb4, ub, g, b, sw, sb):
  return pl.pallas_call(
      _update_score_kernel,
      grid=(N_NODES // BM,),
      in_specs=list(_UPD_IN_SPECS) + [
          pl.BlockSpec((H, 1), lambda i: (0, 0)),
          pl.BlockSpec((1, 1), lambda i: (0, 0)),
      ],
      out_specs=[
          pl.BlockSpec((BM, H), lambda i: (i, 0)),
          pl.BlockSpec((BM, 1), lambda i: (i, 0)),
      ],
      out_shape=[
          jax.ShapeDtypeStruct((N_NODES, H), jnp.float32),
          jax.ShapeDtypeStruct((N_NODES, 1), jnp.float32),
      ],
  )(h, ag, wt, wb4, ub, g, b, sw, sb)


def _sc_edge_body(a_hbm, c_hbm, em_hbm, src_hbm, dst_hbm, zeros_hbm, out_hbm,
                  src_buf, dst_buf, gsrc_buf, gdst_buf,
                  a_buf, c_buf, e_buf, aggr_sh, sem):
  cid = lax.axis_index("c")
  sid = lax.axis_index("s")
  row_base = pl.multiple_of(sid * RPT, 8)

  for kk in range(CHUNKS_PER_CORE):
    k = kk * NC + cid
    koff = pl.multiple_of(k * N_NODES, 8)
    eoff = pl.multiple_of(k * N_EDGES, 8)

    pltpu.sync_copy(zeros_hbm.at[pl.ds(row_base, RPT)],
                    aggr_sh.at[pl.ds(row_base, RPT)])

    @pl.when(sid == 0)
    def _zero_tail():
      pltpu.sync_copy(zeros_hbm.at[pl.ds(RPT * NS, RTAIL)],
                      aggr_sh.at[pl.ds(RPT * NS, RTAIL)])

    plsc.subcore_barrier()

    def batch_body(g, carry):
      ebase = pl.multiple_of(sid * EDGES_PER_TILE + g * EB, 8)
      pltpu.sync_copy(src_hbm.at[pl.ds(ebase, EB)], src_buf)
      pltpu.sync_copy(dst_hbm.at[pl.ds(ebase, EB)], dst_buf)
      for j in range(EB // LANES):
        sl = pl.ds(j * LANES, LANES)
        gsrc_buf[sl] = src_buf[sl] + koff
        gdst_buf[sl] = dst_buf[sl] + koff
      cp_a = pltpu.async_copy(a_hbm.at[gsrc_buf], a_buf, sem)
      cp_c = pltpu.async_copy(c_hbm.at[gdst_buf], c_buf, sem)
      cp_e = pltpu.async_copy(
          em_hbm.at[pl.ds(pl.multiple_of(eoff + ebase, 8), EB)], e_buf, sem)
      cp_a.wait()
      cp_c.wait()
      cp_e.wait()

      def row_body(r, c2):
        for j in range(CHUNK // LANES):
          sl = pl.ds(j * LANES, LANES)
          v = a_buf[r, sl] + c_buf[r, sl] + e_buf[r, sl]
          a_buf[r, sl] = jnp.maximum(v, 0.0)
        return c2

      lax.fori_loop(0, EB, row_body, 0)
      pltpu.sync_copy(a_buf, aggr_sh.at[dst_buf], add=True)
      return carry

    lax.fori_loop(0, NBATCH, batch_body, 0)
    plsc.subcore_barrier()

    pltpu.sync_copy(aggr_sh.at[pl.ds(row_base, RPT)],
                    out_hbm.at[pl.ds(pl.multiple_of(koff + row_base, 8), RPT)])

    @pl.when(sid == 0)
    def _flush_tail():
      pltpu.sync_copy(
          aggr_sh.at[pl.ds(RPT * NS, RTAIL)],
          out_hbm.at[pl.ds(pl.multiple_of(koff + RPT * NS, 8), RTAIL)])

    plsc.subcore_barrier()


@functools.cache
def _sc_edge_pass_fn():
  return pl.kernel(
      _sc_edge_body,
      out_type=jax.ShapeDtypeStruct((NCHUNK * N_NODES, CHUNK), jnp.float32),
      mesh=plsc.VectorSubcoreMesh(core_axis_name="c", subcore_axis_name="s",
                                  num_cores=NC, num_subcores=NS),
      scratch_types=[
        pltpu.VMEM((EB,), jnp.int32),
          pltpu.VMEM((EB,), jnp.int32),
          pltpu.VMEM((EB,), jnp.int32),
          pltpu.VMEM((EB,), jnp.int32),
          pltpu.VMEM((EB, CHUNK), jnp.float32),
          pltpu.VMEM((EB, CHUNK), jnp.float32),
          pltpu.VMEM((EB, CHUNK), jnp.float32),
          pltpu.VMEM_SHARED((N_NODES, CHUNK), jnp.float32),
          pltpu.SemaphoreType.DMA,
      ],
  )


def kernel(node_features, edge_index, edge_features, node_W, node_b,
           edge_W, edge_b,
           msg_W0, msg_b0, upd_W0, upd_b0, ln_g0, ln_b0,
           msg_W1, msg_b1, upd_W1, upd_b1, ln_g1, ln_b1,
           msg_W2, msg_b2, upd_W2, upd_b2, ln_g2, ln_b2,
           score_W, score_b):
  src = edge_index[0]
  dst = edge_index[1]
  zeros_chunk = jnp.zeros((N_NODES, CHUNK), jnp.float32)

  h = _node_proj(node_features, node_W, node_b.reshape(1, H))

  msg_Ws = (msg_W0, msg_W1, msg_W2)
  ems = _edge_msg(edge_features, edge_W, edge_b.reshape(1, H),
                  msg_W0[H:2 * H], msg_W1[H:2 * H], msg_W2[H:2 * H])
  ems = [em.reshape(NCHUNK * N_EDGES, CHUNK) for em in ems]

  layers = [
      (msg_W0, msg_b0, upd_W0, upd_b0, ln_g0, ln_b0),
      (msg_W1, msg_b1, upd_W1, upd_b1, ln_g1, ln_b1),
      (msg_W2, msg_b2, upd_W2, upd_b2, ln_g2, ln_b2),
  ]
  scores = None
  for l, (mW, mb, uW, ub, g, b) in enumerate(layers):
    a4, c4 = _ac_proj(h, mW[:H], mW[2 * H:], mb.reshape(1, H))
    aggr = _sc_edge_pass_fn()(
        a4.reshape(NCHUNK * N_NODES, CHUNK),
        c4.reshape(NCHUNK * N_NODES, CHUNK),
        ems[l], src, dst, zeros_chunk)
    ag4 = aggr.reshape(NCHUNK, N_NODES, CHUNK)
    wt = uW[:H]
    wb4 = uW[H:].reshape(NCHUNK, CHUNK, H)
    if l < 2:
      h = _update(h, ag4, wt, wb4, ub.reshape(1, H),
                  g.reshape(1, H), b.reshape(1, H))
    else:
      h, scores = _update_score(h, ag4, wt, wb4, ub.reshape(1, H),
                                g.reshape(1, H), b.reshape(1, H),
                                score_W, score_b.reshape(1, 1))
  return (h, scores)


# speedup vs baseline: 2.6634x; 1.3499x over previous
import functools

import numpy as np

import jax
import jax.numpy as jnp
from jax import lax
from jax.experimental import pallas as pl
from jax.experimental.pallas import tpu as pltpu
from jax.experimental.pallas import tpu_sc as plsc

N_NODES = 10000
N_EDGES = 160000
D_IN = 256
H = 512

NC = 2
NS = 16
LANES = 16
CHUNK = 128
NCHUNK = H // CHUNK
CHUNKS_PER_CORE = NCHUNK // NC
EDGES_PER_TILE = N_EDGES // NS
EB = 80
NBATCH = EDGES_PER_TILE // EB
RPT = 624
RTAIL = N_NODES - RPT * NS

BM = 400
BE = 400

_EPS = 1e-5

_COLPERM = np.concatenate(
    [np.concatenate([np.arange(32 * g, 32 * g + 32, 2),
                     np.arange(32 * g + 1, 32 * g + 32, 2)])
     for g in range(CHUNK // 32)])
_COLPERM_GLOBAL = np.concatenate([k * CHUNK + _COLPERM for k in range(NCHUNK)])


def _relu_mm_kernel(x_ref, w_ref, b_ref, o_ref):
  o_ref[:] = jnp.maximum(
      jnp.dot(x_ref[:], w_ref[:], preferred_element_type=jnp.float32)
      + b_ref[:], 0.0)


def _node_proj(x, w, b):
  return pl.pallas_call(
      _relu_mm_kernel,
      grid=(N_NODES // BM,),
      in_specs=[
          pl.BlockSpec((BM, D_IN), lambda i: (i, 0)),
          pl.BlockSpec((D_IN, H), lambda i: (0, 0)),
          pl.BlockSpec((1, H), lambda i: (0, 0)),
      ],
      out_specs=pl.BlockSpec((BM, H), lambda i: (i, 0)),
      out_shape=jax.ShapeDtypeStruct((N_NODES, H), jnp.float32),
  )(x, w, b)


def _edge_msg_kernel(ef_ref, ew_ref, eb_ref, w0_ref, w1_ref, w2_ref,
                     o0_ref, o1_ref, o2_ref):
  e = jnp.maximum(
      jnp.dot(ef_ref[:], ew_ref[:], preferred_element_type=jnp.float32)
      + eb_ref[:], 0.0)
  for w_ref, o_ref in ((w0_ref, o0_ref), (w1_ref, o1_ref), (w2_ref, o2_ref)):
    em = jnp.dot(e, w_ref[:], preferred_element_type=jnp.float32)
    for k in range(NCHUNK):
      o_ref[k] = em[:, k * CHUNK:(k + 1) * CHUNK]


def _edge_msg(ef, ew, eb, we0, we1, we2):
  out = jax.ShapeDtypeStruct((NCHUNK, N_EDGES, CHUNK), jnp.float32)
  return pl.pallas_call(
      _edge_msg_kernel,
      grid=(N_EDGES // BE,),
      in_specs=[
          pl.BlockSpec((BE, D_IN), lambda i: (i, 0)),
          pl.BlockSpec((D_IN, H), lambda i: (0, 0)),
          pl.BlockSpec((1, H), lambda i: (0, 0)),
          pl.BlockSpec((H, H), lambda i: (0, 0)),
          pl.BlockSpec((H, H), lambda i: (0, 0)),
          pl.BlockSpec((H, H), lambda i: (0, 0)),
      ],
      out_specs=[pl.BlockSpec((NCHUNK, BE, CHUNK), lambda i: (0, i, 0))] * 3,
      out_shape=[out, out, out],
  )(ef, ew, eb, we0, we1, we2)


def _ac_kernel(h_ref, wj_ref, wi_ref, mb_ref, a_ref, c_ref):
  a = jnp.dot(h_ref[:], wj_ref[:], preferred_element_type=jnp.float32) + mb_ref[:]
  c = jnp.dot(h_ref[:], wi_ref[:], preferred_element_type=jnp.float32)
  for k in range(NCHUNK):
    sl = slice(k * CHUNK, (k + 1) * CHUNK)
    a_ref[k] = a[:, sl]
    c_ref[k] = c[:, sl]


def _ac_proj(h, wj, wi, mb):
  out = jax.ShapeDtypeStruct((NCHUNK, N_NODES, CHUNK), jnp.float32)
  return pl.pallas_call(
      _ac_kernel,
      grid=(N_NODES // BM,),
      in_specs=[
          pl.BlockSpec((BM, H), lambda i: (i, 0)),
          pl.BlockSpec((H, H), lambda i: (0, 0)),
          pl.BlockSpec((H, H), lambda i: (0, 0)),
          pl.BlockSpec((1, H), lambda i: (0, 0)),
      ],
      out_specs=[pl.BlockSpec((NCHUNK, BM, CHUNK), lambda i: (0, i, 0))] * 2,
      out_shape=[out, out],
  )(h, wj, wi, mb)


def _layer_norm(u, g_ref, b_ref):
  mu = jnp.mean(u, axis=-1, keepdims=True)
  var = jnp.mean((u - mu) ** 2, axis=-1, keepdims=True)
  return (u - mu) / jnp.sqrt(var + _EPS) * g_ref[:] + b_ref[:]


def _update_kernel(h_ref, ag_ref, wt_ref, wb_ref, ub_ref, g_ref, b_ref, o_ref):
  acc = jnp.dot(h_ref[:], wt_ref[:], preferred_element_type=jnp.float32) + ub_ref[:]
  for k in range(NCHUNK):
    acc += jnp.dot(ag_ref[k], wb_ref[k], preferred_element_type=jnp.float32)
  o_ref[:] = _layer_norm(jnp.maximum(acc, 0.0), g_ref, b_ref)


def _update_score_kernel(h_ref, ag_ref, wt_ref, wb_ref, ub_ref, g_ref, b_ref,
                         sw_ref, sb_ref, o_ref, s_ref):
  acc = jnp.dot(h_ref[:], wt_ref[:], preferred_element_type=jnp.float32) + ub_ref[:]
  for k in range(NCHUNK):
    acc += jnp.dot(ag_ref[k], wb_ref[k], preferred_element_type=jnp.float32)
  hn = _layer_norm(jnp.maximum(acc, 0.0), g_ref, b_ref)
  o_ref[:] = hn
  s_ref[:] = jnp.dot(hn, sw_ref[:], preferred_element_type=jnp.float32) + sb_ref[:]


_UPD_IN_SPECS = [
    pl.BlockSpec((BM, H), lambda i: (i, 0)),
    pl.BlockSpec((NCHUNK, BM, CHUNK), lambda i: (0, i, 0)),
    pl.BlockSpec((H, H), lambda i: (0, 0)),
    pl.BlockSpec((NCHUNK, CHUNK, H), lambda i: (0, 0, 0)),
    pl.BlockSpec((1, H), lambda i: (0, 0)),
    pl.BlockSpec((1, H), lambda i: (0, 0)),
    pl.BlockSpec((1, H), lambda i: (0, 0)),
]


def _update(h, ag, wt, wb4, ub, g, b):
  return pl.pallas_call(
      _update_kernel,
      grid=(N_NODES // BM,),
      in_specs=list(_UPD_IN_SPECS),
      out_specs=pl.BlockSpec((BM, H), lambda i: (i, 0)),
      out_shape=jax.ShapeDtypeStruct((N_NODES, H), jnp.float32),
  )(h, ag, wt, wb4, ub, g, b)


def _update_score(h, ag, wt, wb4, ub, g, b, sw, sb):
  return pl.pallas_call(
      _update_score_kernel,
      grid=(N_NODES // BM,),
      in_specs=list(_UPD_IN_SPECS) + [
          pl.BlockSpec((H, 1), lambda i: (0, 0)),
          pl.BlockSpec((1, 1), lambda i: (0, 0)),
      ],
      out_specs=[
          pl.BlockSpec((BM, H), lambda i: (i, 0)),
          pl.BlockSpec((BM, 1), lambda i: (i, 0)),
      ],
      out_shape=[
          jax.ShapeDtypeStruct((N_NODES, H), jnp.float32),
          jax.ShapeDtypeStruct((N_NODES, 1), jnp.float32),
      ],
  )(h, ag, wt, wb4, ub, g, b, sw, sb)


def _sc_edge_body(a_hbm, c_hbm, em_hbm, src_hbm, dst_hbm, out_hbm,
                  sr0, dr0, gsrc0, gdst0, draw0,
                  sr1, dr1, gsrc1, gdst1, draw1,
                  e_buf, aggr_sh, semg0, semg1, semi0, semi1, seme):
  pl.run_scoped(
      functools.partial(
          _sc_edge_inner, a_hbm, c_hbm, em_hbm, src_hbm, dst_hbm,
          out_hbm, sr0, dr0, gsrc0, gdst0, draw0,
          sr1, dr1, gsrc1, gdst1, draw1, e_buf, aggr_sh,
          semg0, semg1, semi0, semi1, seme),
      a0=pltpu.VMEM((EB, CHUNK), jnp.float32),
      a1=pltpu.VMEM((EB, CHUNK), jnp.float32),
      c_buf=pltpu.VMEM((EB, CHUNK), jnp.float32),
  )


_HOPS = [(h * EB, EB) for h in range(RPT // EB)] + [(RPT - RPT % EB, RPT % EB)]


def _sc_edge_inner(a_hbm, c_hbm, em_hbm, src_hbm, dst_hbm, out_hbm,
                   sr0, dr0, gsrc0, gdst0, draw0,
                   sr1, dr1, gsrc1, gdst1, draw1,
                   e_buf, aggr_sh, semg0, semg1, semi0, semi1, seme,
                   a0, a1, c_buf):
  cid = lax.axis_index("c")
  sid = lax.axis_index("s")
  row_base = pl.multiple_of(sid * RPT, 8)
  ep_base = pl.multiple_of(sid * EDGES_PER_TILE, 8)

  bufs = ((sr0, dr0, gsrc0, gdst0, draw0, a0, semg0, semi0),
          (sr1, dr1, gsrc1, gdst1, draw1, a1, semg1, semi1))

  def chunk_body(kk, carry0):
    k = kk * NC + cid
    koff = pl.multiple_of(k * N_NODES, 8)
    eoff = pl.multiple_of(k * N_EDGES, 8)

    def zero_row(r, cz):
      for j in range(CHUNK // LANES):
        a0[r, pl.ds(j * LANES, LANES)] = jnp.zeros((LANES,), jnp.float32)
      return cz

    lax.fori_loop(0, EB, zero_row, 0)
    for (hoff, hrows) in _HOPS:
      pltpu.sync_copy(
          a0.at[pl.ds(0, hrows)],
          aggr_sh.at[pl.ds(pl.multiple_of(row_base + hoff, 8), hrows)])

    @pl.when(sid == 0)
    def _zero_tail():
      pltpu.sync_copy(a0.at[pl.ds(0, RTAIL)],
                      aggr_sh.at[pl.ds(RPT * NS, RTAIL)])

    plsc.subcore_barrier()

    def idx_issue(s, b):
      sr, dr = bufs[s][0], bufs[s][1]
      semi = bufs[s][7]
      off = pl.multiple_of(ep_base + b * EB, 8)
      pltpu.async_copy(src_hbm.at[pl.ds(off, EB)], sr, semi)
      pltpu.async_copy(dst_hbm.at[pl.ds(off, EB)], dr, semi)

    def ce_issue(s, b):
      gdst = bufs[s][3]
      pltpu.async_copy(c_hbm.at[gdst], c_buf, seme)
      off = pl.multiple_of(eoff + ep_base + b * EB, 8)
      pltpu.async_copy(em_hbm.at[pl.ds(off, EB)], e_buf, seme)

    def fire(s, b):
      sr, dr, gsrc, gdst, draw, a_b, semg, semi = bufs[s]
      pltpu.make_async_copy(src_hbm.at[pl.ds(0, EB)], sr, semi).wait()
      pltpu.make_async_copy(src_hbm.at[pl.ds(0, EB)], dr, semi).wait()
      for j in range(EB // LANES):
        sl = pl.ds(j * LANES, LANES)
        gsrc[sl] = sr[sl] + koff
        d = dr[sl]
        draw[sl] = d
        gdst[sl] = d + koff

      idx_issue(s, jnp.minimum(b + 2, NBATCH - 1))

      pltpu.async_copy(a_hbm.at[gsrc], a_b, semg)

    def finish(s, b, sn):
      sr, dr, gsrc, gdst, draw, a_b, semg, semi = bufs[s]
      pltpu.make_async_copy(em_hbm.at[pl.ds(0, EB)], a_b, semg).wait()
      pltpu.make_async_copy(em_hbm.at[pl.ds(0, EB)], c_buf, seme).wait()
      pltpu.make_async_copy(em_hbm.at[pl.ds(0, EB)], e_buf, seme).wait()

      def row_body(r, c2):
        for j in range(CHUNK // LANES):
          sl = pl.ds(j * LANES, LANES)
          v = a_b[r, sl] + c_buf[r, sl] + e_buf[r, sl]
          a_b[r, sl] = jnp.maximum(v, 0.0)
        return c2

      lax.fori_loop(0, EB, row_body, 0)
      ce_issue(sn, jnp.minimum(b + 1, NBATCH - 1))
      pltpu.sync_copy(a_b, aggr_sh.at[draw], add=True)

    idx_issue(0, 0)
    idx_issue(1, 1)
    fire(0, 0)
    ce_issue(0, 0)

    def pair_body(g2, carry):
      fire(1, 2 * g2 + 1)
      finish(0, 2 * g2, 1)
      fire(0, 2 * g2 + 2)
      finish(1, 2 * g2 + 1, 0)
      return carry

    lax.fori_loop(0, (NBATCH - 1) // 2, pair_body, 0)
    finish(0, NBATCH - 1, 1)
    for s in range(2):
      sr, dr = bufs[s][0], bufs[s][1]
      semi = bufs[s][7]
      pltpu.make_async_copy(src_hbm.at[pl.ds(0, EB)], sr, semi).wait()
      pltpu.make_async_copy(src_hbm.at[pl.ds(0, EB)], dr, semi).wait()
    pltpu.make_async_copy(em_hbm.at[pl.ds(0, EB)], c_buf, seme).wait()
    pltpu.make_async_copy(em_hbm.at[pl.ds(0, EB)], e_buf, seme).wait()
    plsc.subcore_barrier()

    for (hoff, hrows) in _HOPS:
      pltpu.sync_copy(
          aggr_sh.at[pl.ds(pl.multiple_of(row_base + hoff, 8), hrows)],
          a0.at[pl.ds(0, hrows)])
      pltpu.sync_copy(
          a0.at[pl.ds(0, hrows)],
          out_hbm.at[pl.ds(pl.multiple_of(koff + row_base + hoff, 8), hrows)])

    @pl.when(sid == 0)
    def _flush_tail():
      pltpu.sync_copy(aggr_sh.at[pl.ds(RPT * NS, RTAIL)],
                      a0.at[pl.ds(0, RTAIL)])
      pltpu.sync_copy(
          a0.at[pl.ds(0, RTAIL)],
          out_hbm.at[pl.ds(pl.multiple_of(koff + RPT * NS, 8), RTAIL)])

    plsc.subcore_barrier()
    return carry0

  lax.fori_loop(0, CHUNKS_PER_CORE, chunk_body, 0)


@functools.cache
def _sc_edge_pass_fn():
  return pl.kernel(
      _sc_edge_body,
      out_type=jax.ShapeDtypeStruct((NCHUNK * N_NODES, CHUNK), jnp.float32),
      mesh=plsc.VectorSubcoreMesh(core_axis_name="c", subcore_axis_name="s",
                                  num_cores=NC, num_subcores=NS),
      scratch_types=(
          [pltpu.VMEM((EB,), jnp.int32)] * 10
          + [pltpu.VMEM((EB, CHUNK), jnp.float32)]
          + [pltpu.VMEM_SHARED((N_NODES, CHUNK), jnp.float32)]
          + [pltpu.SemaphoreType.DMA] * 5),
  )


def kernel(node_features, edge_index, edge_features, node_W, node_b,
           edge_W, edge_b,
           msg_W0, msg_b0, upd_W0, upd_b0, ln_g0, ln_b0,
           msg_W1, msg_b1, upd_W1, upd_b1, ln_g1, ln_b1,
           msg_W2, msg_b2, upd_W2, upd_b2, ln_g2, ln_b2,
           score_W, score_b):
  src = edge_index[0]
  dst = edge_index[1]

  h = _node_proj(node_features, node_W, node_b.reshape(1, H))

  msg_Ws = (msg_W0, msg_W1, msg_W2)
  ems = _edge_msg(edge_features, edge_W, edge_b.reshape(1, H),
                  msg_W0[H:2 * H], msg_W1[H:2 * H], msg_W2[H:2 * H])
  ems = [em.reshape(NCHUNK * N_EDGES, CHUNK) for em in ems]

  layers = [
      (msg_W0, msg_b0, upd_W0, upd_b0, ln_g0, ln_b0),
      (msg_W1, msg_b1, upd_W1, upd_b1, ln_g1, ln_b1),
      (msg_W2, msg_b2, upd_W2, upd_b2, ln_g2, ln_b2),
  ]
  scores = None
  for l, (mW, mb, uW, ub, g, b) in enumerate(layers):
    a4, c4 = _ac_proj(h, mW[:H], mW[2 * H:], mb.reshape(1, H))
    aggr = _sc_edge_pass_fn()(
        a4.reshape(NCHUNK * N_NODES, CHUNK),
        c4.reshape(NCHUNK * N_NODES, CHUNK),
        ems[l], src, dst)
    ag4 = aggr.reshape(NCHUNK, N_NODES, CHUNK)
    wt = uW[:H]
    wb4 = uW[H:].reshape(NCHUNK, CHUNK, H)
    if l < 2:
      h = _update(h, ag4, wt, wb4, ub.reshape(1, H),
                  g.reshape(1, H), b.reshape(1, H))
    else:
      h, scores = _update_score(h, ag4, wt, wb4, ub.reshape(1, H),
                                g.reshape(1, H), b.reshape(1, H),
                                score_W, score_b.reshape(1, 1))
  return (h, scores)
